# baseline (speedup 1.0000x reference)
import functools

import jax
import jax.numpy as jnp
from jax import lax
from jax.experimental import pallas as pl
from jax.experimental.pallas import tpu as pltpu
from jax.experimental.pallas import tpu_sc as plsc

N = 10000
E = 320000
G = 64
D = 128
T = 10

NPAD = 10240
ROWS_PER_TILE = NPAD // 16
EPT = 10240
DCH = 64
ECH = EPT // DCH
CH = 16
EPAD = EPT * 32
EPT2 = EPAD // 16
NCHT = EPT2 // CH
SEC = 8
NSEC = NCHT // SEC
NHALF = NPAD // 2
TRASH = NHALF
ACCR = NHALF + 8
YSP = 10016
GPAD = 72
PCH = 8
PNCH = NPAD // 64

BN_SCALE = 0.9999950000374996

_mesh = plsc.VectorSubcoreMesh(core_axis_name="c", subcore_axis_name="s")
_f32 = jnp.float32
_i32 = jnp.int32


@functools.partial(
    pl.kernel,
    out_type=jax.ShapeDtypeStruct((2, NPAD, D), _f32),
    mesh=_mesh,
    scratch_types=[
        pltpu.VMEM((ECH, DCH), _i32),
        pltpu.VMEM((DCH, D), _f32),
        pltpu.VMEM_SHARED((NPAD, D), _f32),
    ],
)
def _deg_kernel(dst2_hbm, ones_hbm, zd_hbm, out, dst_v, ones_v, deg_sh):
    c = lax.axis_index("c")
    s = lax.axis_index("s")
    wid = c * 16 + s
    base = s * ROWS_PER_TILE
    pltpu.sync_copy(zd_hbm, deg_sh.at[pl.ds(base, ROWS_PER_TILE)])
    pltpu.sync_copy(ones_hbm, ones_v)
    pltpu.sync_copy(dst2_hbm.at[pl.ds(wid * ECH, ECH)], dst_v)
    plsc.subcore_barrier()

    def body(j, carry):
        pltpu.sync_copy(ones_v, deg_sh.at[dst_v.at[j]], add=True)
        return carry

    lax.fori_loop(0, ECH, body, 0)
    plsc.subcore_barrier()
    pltpu.sync_copy(deg_sh.at[pl.ds(base, ROWS_PER_TILE)],
                    out.at[c, pl.ds(base, ROWS_PER_TILE)])


@functools.partial(
    pl.kernel,
    out_type=jax.ShapeDtypeStruct((NPAD, D), _f32),
    mesh=_mesh,
    scratch_types=[
        pltpu.VMEM((4, SEC, CH), _i32),
        pltpu.VMEM((2, CH, D), _f32),
        pltpu.VMEM_SHARED((YSP, D), _f32),
        pltpu.VMEM_SHARED((ACCR, D), _f32),
        pltpu.SemaphoreType.DMA,
        pltpu.SemaphoreType.DMA,
        pltpu.SemaphoreType.DMA,
        pltpu.SemaphoreType.DMA,
        pltpu.SemaphoreType.DMA,
        pltpu.SemaphoreType.DMA,
    ],
)
def _edge_kernel(y_hbm, src2_hbm, dstr_hbm, z_hbm, out,
                 idx_v, buf_v, y_sp, acc_sh, gs0, gs1, ss0, ss1, isA, isB):
    c = lax.axis_index("c")
    s = lax.axis_index("s")
    gsems = (gs0, gs1)
    ssems = (ss0, ss1)

    @pl.when(s < 4)
    def _():
        pltpu.sync_copy(y_hbm.at[pl.ds(s * 2504, 2504)],
                        y_sp.at[pl.ds(s * 2504, 2504)])

    pltpu.sync_copy(z_hbm.at[pl.ds(0, NHALF // 16)],
                    acc_sh.at[pl.ds(s * (NHALF // 16), NHALF // 16)])
    crow = s * NCHT

    def ldsec(sec, sp, dp, sem):
        pltpu.async_copy(src2_hbm.at[pl.ds(crow + sec * SEC, SEC)],
                         idx_v.at[sp], sem)
        pltpu.async_copy(dstr_hbm.at[c, pl.ds(crow + sec * SEC, SEC)],
                         idx_v.at[dp], sem)

    def ldwait(sp, dp, sem):
        pltpu.make_async_copy(src2_hbm.at[pl.ds(crow, SEC)],
                              idx_v.at[sp], sem).wait()
        pltpu.make_async_copy(dstr_hbm.at[c, pl.ds(crow, SEC)],
                              idx_v.at[dp], sem).wait()

    def gather(sp, k, b):
        pltpu.async_copy(y_sp.at[idx_v.at[sp, k]], buf_v.at[b], gsems[b])

    def gwait(b):
        pltpu.make_async_copy(y_sp.at[idx_v.at[0, 0]], buf_v.at[b],
                              gsems[b]).wait()

    def scat(dp, k, b):
        pltpu.async_copy(buf_v.at[b], acc_sh.at[idx_v.at[dp, k]], ssems[b],
                         add=True)

    def swait(b):
        pltpu.make_async_copy(buf_v.at[b], acc_sh.at[idx_v.at[2, 0]],
                              ssems[b]).wait()

    ldsec(0, 0, 2, isA)
    ldwait(0, 2, isA)
    ldsec(1, 1, 3, isB)
    ldwait(1, 3, isB)
    plsc.subcore_barrier()
    gather(0, 0, 0)

    def body(j, carry):
        jpos = j > 0
        jneg = j < (NSEC // 2 - 1)
        for k in range(SEC):
            b = k % 2
            gwait(b)
            scat(2, k, b)
            if k == 0:
                @pl.when(jpos)
                def _():
                    swait(1)
                    ldsec(2 * j + 1, 1, 3, isB)
            else:
                swait(1 - b)
            if k < SEC - 1:
                gather(0, k + 1, 1 - b)
            else:
                @pl.when(jpos)
                def _():
                    ldwait(1, 3, isB)
                gather(1, 0, 1 - b)
        for k in range(SEC):
            b = k % 2
            gwait(b)
            scat(3, k, b)
            swait(1 - b)
            if k == 0:
                @pl.when(jneg)
                def _():
                    ldsec(2 * j + 2, 0, 2, isA)
            if k < SEC - 1:
                gather(1, k + 1, 1 - b)
            else:
                @pl.when(jneg)
                def _():
                    ldwait(0, 2, isA)
                    gather(0, 0, 1 - b)
        return carry

    lax.fori_loop(0, NSEC // 2, body, 0)
    swait(1)
    plsc.subcore_barrier()
    pltpu.sync_copy(acc_sh.at[pl.ds(s * 320, 320)],
                    out.at[pl.ds(c * NHALF + s * 320, 320)])


@functools.partial(
    pl.kernel,
    out_type=jax.ShapeDtypeStruct((2, GPAD, D), _f32),
    mesh=_mesh,
    scratch_types=[
        pltpu.VMEM((PCH, 64), _i32),
        pltpu.VMEM((64, D), _f32),
        pltpu.VMEM_SHARED((GPAD, D), _f32),
    ],
)
def _pool_kernel(h_hbm, b2_hbm, z_hbm, out, idx_v, buf, pool_sh):
    c = lax.axis_index("c")
    s = lax.axis_index("s")
    wid = c * 16 + s

    @pl.when(s == 0)
    def _():
        pltpu.sync_copy(z_hbm.at[pl.ds(0, GPAD)], pool_sh)

    pltpu.sync_copy(b2_hbm.at[pl.ds(wid * PCH, PCH)], idx_v)
    plsc.subcore_barrier()
    for j in range(PCH):
        @pl.when(wid * PCH + j < PNCH)
        def _():
            pltpu.sync_copy(h_hbm.at[pl.ds((wid * PCH + j) * 64, 64)], buf)
            pltpu.sync_copy(buf, pool_sh.at[idx_v.at[j]], add=True)
    plsc.subcore_barrier()

    @pl.when(s == 0)
    def _():
        pltpu.sync_copy(pool_sh, out.at[c])


_R = 1024


def _ya_body(x_ref, d0_ref, d1_ref, w_ref, y_ref, dis_ref):
    deg = d0_ref[0] + d1_ref[0]
    dis = lax.rsqrt(1.0 + deg)
    y_ref[...] = jnp.dot(x_ref[...], w_ref[...],
                         preferred_element_type=_f32) * dis
    dis_ref[...] = dis


_ya = pl.pallas_call(
    _ya_body,
    grid=(NPAD // _R,),
    in_specs=[
        pl.BlockSpec((_R, D), lambda i: (i, 0)),
        pl.BlockSpec((1, _R, D), lambda i: (0, i, 0)),
        pl.BlockSpec((1, _R, D), lambda i: (1, i, 0)),
        pl.BlockSpec((D, D), lambda i: (0, 0)),
    ],
    out_specs=[pl.BlockSpec((_R, D), lambda i: (i, 0)),
               pl.BlockSpec((_R, D), lambda i: (i, 0))],
    out_shape=[jax.ShapeDtypeStruct((NPAD, D), _f32),
               jax.ShapeDtypeStruct((NPAD, D), _f32)],
)


def _yb_body(a_ref, yp_ref, dis_ref, b_ref, g_ref, be_ref, w_ref, y_ref):
    dis = dis_ref[...]
    conv = (a_ref[...] + yp_ref[...]) * dis + b_ref[...]
    h = jnp.maximum(conv * (g_ref[...] * BN_SCALE) + be_ref[...], 0.0)
    y_ref[...] = jnp.dot(h, w_ref[...], preferred_element_type=_f32) * dis


_yb = pl.pallas_call(
    _yb_body,
    grid=(NPAD // _R,),
    in_specs=[
        pl.BlockSpec((_R, D), lambda i: (i, 0)),
        pl.BlockSpec((_R, D), lambda i: (i, 0)),
        pl.BlockSpec((_R, D), lambda i: (i, 0)),
        pl.BlockSpec((1, D), lambda i: (0, 0)),
        pl.BlockSpec((1, D), lambda i: (0, 0)),
        pl.BlockSpec((1, D), lambda i: (0, 0)),
        pl.BlockSpec((D, D), lambda i: (0, 0)),
    ],
    out_specs=pl.BlockSpec((_R, D), lambda i: (i, 0)),
    out_shape=jax.ShapeDtypeStruct((NPAD, D), _f32),
)


def _yc_body(a_ref, yp_ref, dis_ref, b_ref, g_ref, be_ref, h_ref):
    conv = (a_ref[...] + yp_ref[...]) * dis_ref[...] + b_ref[...]
    h_ref[...] = conv * (g_ref[...] * BN_SCALE) + be_ref[...]


_yc = pl.pallas_call(
    _yc_body,
    grid=(NPAD // _R,),
    in_specs=[
        pl.BlockSpec((_R, D), lambda i: (i, 0)),
        pl.BlockSpec((_R, D), lambda i: (i, 0)),
        pl.BlockSpec((_R, D), lambda i: (i, 0)),
        pl.BlockSpec((1, D), lambda i: (0, 0)),
        pl.BlockSpec((1, D), lambda i: (0, 0)),
        pl.BlockSpec((1, D), lambda i: (0, 0)),
    ],
    out_specs=pl.BlockSpec((_R, D), lambda i: (i, 0)),
    out_shape=jax.ShapeDtypeStruct((NPAD, D), _f32),
)


def _out_body(p_ref, w_ref, bo_ref, o_ref):
    p = (p_ref[0] + p_ref[1])[:G]
    z = jnp.dot(p, w_ref[...], preferred_element_type=_f32) + bo_ref[...]
    o_ref[...] = jnp.where(z >= 0, z, 0.1 * z)


_outk = pl.pallas_call(
    _out_body,
    in_specs=[
        pl.BlockSpec((2, GPAD, D), lambda: (0, 0, 0)),
        pl.BlockSpec((D, D), lambda: (0, 0)),
        pl.BlockSpec((1, D), lambda: (0, 0)),
    ],
    out_specs=pl.BlockSpec((G, D), lambda: (0, 0)),
    out_shape=jax.ShapeDtypeStruct((G, D), _f32),
)


def kernel(x, edge_index, edge_attr, batch,
           W0, b0, g0, be0, W1, b1, g1, be1,
           W2, b2, g2, be2, W3, b3, g3, be3, Wout, bout):
    src = edge_index[0]
    dst = edge_index[1]
    pad_idx = jnp.full((EPAD - E,), N, _i32)
    src2 = jnp.concatenate(
        [src, jnp.zeros((EPAD - E,), _i32)]).reshape(EPAD // CH, CH)
    dst_p = jnp.concatenate([dst, pad_idx])
    dst2 = dst_p.reshape(EPAD // DCH, DCH)
    r0 = jnp.where(dst_p < NHALF, dst_p, TRASH)
    r1 = jnp.where((dst_p >= NHALF) & (dst_p < N), dst_p - NHALF, TRASH)
    dstr = jnp.stack([r0, r1]).reshape(2, EPAD // CH, CH)
    batch2 = jnp.concatenate(
        [batch, jnp.full((32 * PCH * 64 - N,), G, _i32)]).reshape(32 * PCH, 64)
    x_p = jnp.pad(x, ((0, NPAD - N), (0, 0)))

    zeros_rows = jnp.zeros((ROWS_PER_TILE, D), _f32)
    ones_ch = jnp.ones((DCH, D), _f32)

    deg = _deg_kernel(dst2, ones_ch, zeros_rows)
    y, dis = _ya(x_p, deg, deg, W0.T)

    Ws = [W1, W2, W3]
    bs = [b0, b1, b2, b3]
    gs = [g0, g1, g2, g3]
    bes = [be0, be1, be2, be3]
    for l in range(4):
        acc = _edge_kernel(y, src2, dstr, zeros_rows)
        brow = bs[l].reshape(1, D)
        grow = gs[l].reshape(1, D)
        berow = bes[l].reshape(1, D)
        if l < 3:
            y = _yb(acc, y, dis, brow, grow, berow, Ws[l].T)
        else:
            h4 = _yc(acc, y, dis, brow, grow, berow)

    p = _pool_kernel(h4, batch2, zeros_rows)
    w_out = jnp.zeros((D, D), _f32).at[:, :T].set(Wout.T)
    b_out = jnp.zeros((1, D), _f32).at[0, :T].set(bout)
    out = _outk(p, w_out, b_out)
    return out[:, :T]
